# baseline (speedup 1.0000x reference)
import functools

import jax
import jax.numpy as jnp
from jax import lax
from jax.experimental import pallas as pl
from jax.experimental.pallas import tpu as pltpu
from jax.experimental.pallas import tpu_sc as plsc

Q_LEN = 2048
K_LEN = 2048
EMB = 64
ROW128 = K_LEN * EMB // 128
NWORKERS = 32
PAIRS_PER_WORKER = Q_LEN // 2 // NWORKERS


def _sc_body(reva_hbm, revb_hbm, out_hbm, reva_sh, revb_sh, sem_e, sem_o):
    s = lax.axis_index("s")
    c = lax.axis_index("c")
    wid = s * 2 + c

    @pl.when(s == 0)
    def _():
        pltpu.sync_copy(reva_hbm, reva_sh)
        pltpu.sync_copy(revb_hbm, revb_sh)

    plsc.subcore_barrier()

    def pair(p, carry):
        even = pltpu.make_async_copy(
            revb_sh.at[pl.ds(ROW128 - 1 - p, ROW128), :],
            out_hbm.at[2 * p], sem_e)
        odd = pltpu.make_async_copy(
            reva_sh.at[pl.ds(ROW128 - 1 - p, ROW128), :],
            out_hbm.at[2 * p + 1], sem_o)
        even.start()
        odd.start()
        even.wait()
        odd.wait()
        return carry

    lax.fori_loop(wid * PAIRS_PER_WORKER, (wid + 1) * PAIRS_PER_WORKER,
                  pair, 0)


_sc_call = functools.partial(
    pl.kernel,
    out_type=jax.ShapeDtypeStruct((Q_LEN, ROW128, 128), jnp.float32),
    mesh=plsc.VectorSubcoreMesh(core_axis_name="c", subcore_axis_name="s"),
    scratch_types=[
        pltpu.VMEM_SHARED((2 * ROW128, 128), jnp.float32),
        pltpu.VMEM_SHARED((2 * ROW128 - 1, 128), jnp.float32),
        pltpu.SemaphoreType.DMA,
        pltpu.SemaphoreType.DMA,
    ],
)(_sc_body)


def kernel(q, k, rel_emb):
    rev_flat = jnp.flip(rel_emb, axis=0).reshape(-1)
    reva = rev_flat.reshape(2 * ROW128, 128)
    revb = jax.lax.dynamic_slice(rev_flat, (64,),
                                 ((2 * ROW128 - 1) * 128,)).reshape(
                                     2 * ROW128 - 1, 128)
    out = _sc_call(reva, revb)
    return out.reshape(Q_LEN, K_LEN, EMB)
